# baseline (speedup 1.0000x reference)
import functools

import jax
import jax.numpy as jnp
from jax import lax
from jax.experimental import pallas as pl
from jax.experimental.pallas import tpu as pltpu
from jax.experimental.pallas import tpu_sc as plsc

_DIM = 64
_NUM_WORKERS = 32
_BB = 128


def _build(batch: int, seq: int):
    n_bblocks = batch // _BB
    assert n_bblocks == _NUM_WORKERS
    mesh = plsc.VectorSubcoreMesh(core_axis_name="c", subcore_axis_name="s")

    @functools.partial(
        pl.kernel,
        mesh=mesh,
        compiler_params=pltpu.CompilerParams(
            use_tc_tiling_on_sc=False, needs_layout_passes=False),
        out_type=jax.ShapeDtypeStruct(
            (seq, _DIM // 8, n_bblocks, 8, _BB), jnp.float32),
        scratch_types=[
            pltpu.VMEM((_BB, seq), jnp.int32),
            pltpu.VMEM((seq, _BB), jnp.int32),
            pltpu.VMEM((_BB, _DIM), jnp.float32),
            pltpu.VMEM((_BB, _DIM), jnp.float32),
            pltpu.VMEM((_DIM // 8, 8, _BB), jnp.float32),
            pltpu.VMEM((_DIM // 8, 8, _BB), jnp.float32),
            pltpu.SemaphoreType.DMA,
            pltpu.SemaphoreType.DMA,
        ],
    )
    def gather_kernel(idx_hbm, table_hbm, out_hbm, idx_v, idx_t,
                      buf_a, buf_b, slab_a, slab_b, sem_a, sem_b):
        wid = lax.axis_index("s") * 2 + lax.axis_index("c")
        pltpu.sync_copy(idx_hbm.at[pl.ds(wid * _BB, _BB)], idx_v)

        lanes = lax.iota(jnp.int32, 16)

        def tr_idx(t, carry):
            for k in range(_BB // 16):
                rows = lanes + (k * 16)
                vals = plsc.load_gather(idx_v, [rows, jnp.full((16,), 0, jnp.int32) + t])
                idx_t[t, pl.ds(k * 16, 16)] = vals
            return carry

        lax.fori_loop(0, seq, tr_idx, 0)

        def start_t(t, buf, sem):
            pltpu.async_copy(table_hbm.at[idx_t.at[t]], buf, sem)

        def wait_t(buf, sem):
            pltpu.make_async_copy(
                table_hbm.at[pl.ds(0, _BB)], buf, sem).wait()

        def transpose_t(buf, slab):
            def per_d8(d8, carry):
                for j in range(8):
                    for c in range(_BB // 16):
                        vals = plsc.load_gather(
                            buf,
                            [lanes + c * 16,
                             jnp.full((16,), j, jnp.int32) + d8 * 8])
                        slab[d8, j, pl.ds(c * 16, 16)] = vals
                return carry

            lax.fori_loop(0, _DIM // 8, per_d8, 0)

        def out_t(t, slab):
            pltpu.sync_copy(slab, out_hbm.at[t, pl.ds(0, _DIM // 8), wid])

        start_t(0, buf_a, sem_a)

        def body(p, carry):
            start_t(2 * p + 1, buf_b, sem_b)
            wait_t(buf_a, sem_a)
            transpose_t(buf_a, slab_a)
            out_t(2 * p, slab_a)
            start_t(2 * p + 2, buf_a, sem_a)
            wait_t(buf_b, sem_b)
            transpose_t(buf_b, slab_b)
            out_t(2 * p + 1, slab_b)
            return carry

        lax.fori_loop(0, seq // 2 - 1, body, 0)
        start_t(seq - 1, buf_b, sem_b)
        wait_t(buf_a, sem_a)
        transpose_t(buf_a, slab_a)
        out_t(seq - 2, slab_a)
        wait_t(buf_b, sem_b)
        transpose_t(buf_b, slab_b)
        out_t(seq - 1, slab_b)

    return gather_kernel


def kernel(x, table):
    batch, seq = x.shape
    out5 = _build(batch, seq)(x, table)
    return out5.transpose(2, 4, 0, 1, 3).reshape(batch, seq, _DIM)


# speedup vs baseline: 1.2916x; 1.2916x over previous
import functools

import jax
import jax.numpy as jnp
from jax import lax
from jax.experimental import pallas as pl
from jax.experimental.pallas import tpu as pltpu
from jax.experimental.pallas import tpu_sc as plsc

_DIM = 64
_NUM_WORKERS = 32
_MEGA = 8


def _build(batch: int, seq: int):
    rows_per_worker = batch // _NUM_WORKERS
    n_mega = rows_per_worker // _MEGA
    n_pairs = n_mega // 2
    mesh = plsc.VectorSubcoreMesh(core_axis_name="c", subcore_axis_name="s")

    @functools.partial(
        pl.kernel,
        mesh=mesh,
        compiler_params=pltpu.CompilerParams(use_tc_tiling_on_sc=False),
        out_type=jax.ShapeDtypeStruct((batch, seq, _DIM), jnp.float32),
        scratch_types=[
            pltpu.VMEM((rows_per_worker, seq), jnp.int32),
            pltpu.VMEM((_MEGA, seq, 2 * _DIM), jnp.float32),
            pltpu.VMEM((_MEGA, seq, 2 * _DIM), jnp.float32),
            pltpu.SemaphoreType.DMA,
            pltpu.SemaphoreType.DMA,
        ],
    )
    def gather_kernel(idx_hbm, table_hbm, out_hbm, idx_v, buf_a, buf_b, sem_a, sem_b):
        wid = lax.axis_index("s") * 2 + lax.axis_index("c")
        base = wid * rows_per_worker
        pltpu.sync_copy(idx_hbm.at[pl.ds(base, rows_per_worker)], idx_v)

        def start_mega(m, buf, sem):
            for c in range(_MEGA):
                pltpu.async_copy(
                    table_hbm.at[idx_v.at[m * _MEGA + c]],
                    buf.at[c],
                    sem,
                )

        def wait_mega(buf, sem):
            for c in range(_MEGA):
                pltpu.make_async_copy(
                    table_hbm.at[pl.ds(0, seq)], buf.at[c], sem
                ).wait()

        def out_mega(m, buf):
            pltpu.sync_copy(
                buf.at[:, :, pl.ds(0, _DIM)],
                out_hbm.at[pl.ds(base + m * _MEGA, _MEGA)],
            )

        start_mega(0, buf_a, sem_a)

        def body(t, carry):
            start_mega(2 * t + 1, buf_b, sem_b)
            wait_mega(buf_a, sem_a)
            out_mega(2 * t, buf_a)
            start_mega(2 * t + 2, buf_a, sem_a)
            wait_mega(buf_b, sem_b)
            out_mega(2 * t + 1, buf_b)
            return carry

        lax.fori_loop(0, n_pairs - 1, body, 0)
        start_mega(n_mega - 1, buf_b, sem_b)
        wait_mega(buf_a, sem_a)
        out_mega(n_mega - 2, buf_a)
        wait_mega(buf_b, sem_b)
        out_mega(n_mega - 1, buf_b)

    return gather_kernel


def kernel(x, table):
    batch, seq = x.shape
    tpad = jnp.pad(table, ((0, 0), (0, _DIM)))
    return _build(batch, seq)(x, tpad)


# speedup vs baseline: 1.3294x; 1.0293x over previous
import functools

import jax
import jax.numpy as jnp
from jax import lax
from jax.experimental import pallas as pl
from jax.experimental.pallas import tpu as pltpu
from jax.experimental.pallas import tpu_sc as plsc

_DIM = 64
_NUM_WORKERS = 32
_MEGA = 8


def _build(batch: int, seq: int):
    rows_per_worker = batch // _NUM_WORKERS
    n_mega = rows_per_worker // _MEGA
    n_pairs = n_mega // 2
    mesh = plsc.VectorSubcoreMesh(core_axis_name="c", subcore_axis_name="s")

    @functools.partial(
        pl.kernel,
        mesh=mesh,
        compiler_params=pltpu.CompilerParams(use_tc_tiling_on_sc=False),
        out_type=jax.ShapeDtypeStruct((batch, seq, _DIM), jnp.float32),
        scratch_types=[
            pltpu.VMEM((rows_per_worker, seq), jnp.int32),
            pltpu.VMEM((_MEGA, seq, _DIM), jnp.float32),
            pltpu.VMEM((_MEGA, seq, _DIM), jnp.float32),
            pltpu.SemaphoreType.DMA,
            pltpu.SemaphoreType.DMA,
        ],
    )
    def gather_kernel(idx_hbm, table_hbm, out_hbm, idx_v, buf_a, buf_b, sem_a, sem_b):
        wid = lax.axis_index("s") * 2 + lax.axis_index("c")
        base = wid * rows_per_worker
        pltpu.sync_copy(idx_hbm.at[pl.ds(base, rows_per_worker)], idx_v)

        def start_mega(m, buf, sem):
            for c in range(_MEGA):
                pltpu.async_copy(
                    table_hbm.at[idx_v.at[m * _MEGA + c]],
                    buf.at[c],
                    sem,
                )

        def wait_mega(buf, sem):
            pltpu.make_async_copy(
                out_hbm.at[pl.ds(base, _MEGA)], buf, sem
            ).wait()

        def out_mega(m, buf):
            pltpu.sync_copy(
                buf, out_hbm.at[pl.ds(base + m * _MEGA, _MEGA)]
            )

        start_mega(0, buf_a, sem_a)

        def body(t, carry):
            start_mega(2 * t + 1, buf_b, sem_b)
            wait_mega(buf_a, sem_a)
            out_mega(2 * t, buf_a)
            start_mega(2 * t + 2, buf_a, sem_a)
            wait_mega(buf_b, sem_b)
            out_mega(2 * t + 1, buf_b)
            return carry

        lax.fori_loop(0, n_pairs - 1, body, 0)
        start_mega(n_mega - 1, buf_b, sem_b)
        wait_mega(buf_a, sem_a)
        out_mega(n_mega - 2, buf_a)
        wait_mega(buf_b, sem_b)
        out_mega(n_mega - 1, buf_b)

    return gather_kernel


def kernel(x, table):
    batch, seq = x.shape
    tpad = jnp.pad(table, ((0, 0), (0, _DIM))).reshape(-1, _DIM)
    return _build(batch, seq)(x * 2, tpad)


# speedup vs baseline: 1.3295x; 1.0000x over previous
import functools

import jax
import jax.numpy as jnp
from jax import lax
from jax.experimental import pallas as pl
from jax.experimental.pallas import tpu as pltpu
from jax.experimental.pallas import tpu_sc as plsc

_DIM = 64
_NUM_WORKERS = 32
_MEGA = 16


def _build(batch: int, seq: int):
    rows_per_worker = batch // _NUM_WORKERS
    n_mega = rows_per_worker // _MEGA
    n_pairs = n_mega // 2
    mesh = plsc.VectorSubcoreMesh(core_axis_name="c", subcore_axis_name="s")

    @functools.partial(
        pl.kernel,
        mesh=mesh,
        compiler_params=pltpu.CompilerParams(use_tc_tiling_on_sc=False),
        out_type=jax.ShapeDtypeStruct((batch, seq, _DIM), jnp.float32),
        scratch_types=[
            pltpu.VMEM((rows_per_worker, seq), jnp.int32),
            pltpu.VMEM((_MEGA, seq, _DIM), jnp.float32),
            pltpu.VMEM((_MEGA, seq, _DIM), jnp.float32),
            pltpu.SemaphoreType.DMA,
            pltpu.SemaphoreType.DMA,
        ],
    )
    def gather_kernel(idx_hbm, table_hbm, out_hbm, idx_v, buf_a, buf_b, sem_a, sem_b):
        wid = lax.axis_index("s") * 2 + lax.axis_index("c")
        base = wid * rows_per_worker
        pltpu.sync_copy(idx_hbm.at[pl.ds(base, rows_per_worker)], idx_v)

        def start_mega(m, buf, sem):
            for c in range(_MEGA):
                pltpu.async_copy(
                    table_hbm.at[idx_v.at[m * _MEGA + c]],
                    buf.at[c],
                    sem,
                )

        def wait_mega(buf, sem):
            pltpu.make_async_copy(
                out_hbm.at[pl.ds(base, _MEGA)], buf, sem
            ).wait()

        def out_mega(m, buf):
            pltpu.sync_copy(
                buf, out_hbm.at[pl.ds(base + m * _MEGA, _MEGA)]
            )

        start_mega(0, buf_a, sem_a)

        def body(t, carry):
            start_mega(2 * t + 1, buf_b, sem_b)
            wait_mega(buf_a, sem_a)
            out_mega(2 * t, buf_a)
            start_mega(2 * t + 2, buf_a, sem_a)
            wait_mega(buf_b, sem_b)
            out_mega(2 * t + 1, buf_b)
            return carry

        lax.fori_loop(0, n_pairs - 1, body, 0)
        start_mega(n_mega - 1, buf_b, sem_b)
        wait_mega(buf_a, sem_a)
        out_mega(n_mega - 2, buf_a)
        wait_mega(buf_b, sem_b)
        out_mega(n_mega - 1, buf_b)

    return gather_kernel


def kernel(x, table):
    batch, seq = x.shape
    tpad = jnp.pad(table, ((0, 0), (0, _DIM))).reshape(-1, _DIM)
    return _build(batch, seq)(x * 2, tpad)


# speedup vs baseline: 1.3317x; 1.0017x over previous
import functools

import jax
import jax.numpy as jnp
from jax import lax
from jax.experimental import pallas as pl
from jax.experimental.pallas import tpu as pltpu
from jax.experimental.pallas import tpu_sc as plsc

_DIM = 64
_NUM_WORKERS = 32
_MEGA = 8


def _build(batch: int, seq: int):
    rows_per_worker = batch // _NUM_WORKERS
    n_mega = rows_per_worker // _MEGA
    n_pairs = n_mega // 2
    mesh = plsc.VectorSubcoreMesh(core_axis_name="c", subcore_axis_name="s")

    @functools.partial(
        pl.kernel,
        mesh=mesh,
        compiler_params=pltpu.CompilerParams(use_tc_tiling_on_sc=False),
        out_type=jax.ShapeDtypeStruct((batch, seq, _DIM), jnp.float32),
        scratch_types=[
            pltpu.VMEM((rows_per_worker, seq), jnp.int32),
            pltpu.VMEM((_MEGA, seq, _DIM), jnp.float32),
            pltpu.VMEM((_MEGA, seq, _DIM), jnp.float32),
            pltpu.SemaphoreType.DMA,
            pltpu.SemaphoreType.DMA,
        ],
    )
    def gather_kernel(idx_hbm, table_hbm, out_hbm, idx_v, buf_a, buf_b, sem_a, sem_b):
        wid = lax.axis_index("s") * 2 + lax.axis_index("c")
        base = wid * rows_per_worker
        pltpu.sync_copy(idx_hbm.at[pl.ds(base, rows_per_worker)], idx_v)

        def start_mega(m, buf, sem):
            for c in range(_MEGA):
                pltpu.async_copy(
                    table_hbm.at[idx_v.at[m * _MEGA + c]],
                    buf.at[c],
                    sem,
                )

        def wait_mega(buf, sem):
            pltpu.make_async_copy(
                out_hbm.at[pl.ds(base, _MEGA)], buf, sem
            ).wait()

        def out_mega(m, buf):
            pltpu.sync_copy(
                buf, out_hbm.at[pl.ds(base + m * _MEGA, _MEGA)]
            )

        start_mega(0, buf_a, sem_a)

        def body(t, carry):
            start_mega(2 * t + 1, buf_b, sem_b)
            wait_mega(buf_a, sem_a)
            out_mega(2 * t, buf_a)
            start_mega(2 * t + 2, buf_a, sem_a)
            wait_mega(buf_b, sem_b)
            out_mega(2 * t + 1, buf_b)
            return carry

        lax.fori_loop(0, n_pairs - 1, body, 0)
        start_mega(n_mega - 1, buf_b, sem_b)
        wait_mega(buf_a, sem_a)
        out_mega(n_mega - 2, buf_a)
        wait_mega(buf_b, sem_b)
        out_mega(n_mega - 1, buf_b)

    return gather_kernel


def kernel(x, table):
    batch, seq = x.shape
    tpad = jnp.pad(table, ((0, 0), (0, _DIM))).reshape(-1, _DIM)
    return _build(batch, seq)(x * 2, tpad)
